# baseline (speedup 1.0000x reference)
import functools

import jax
import jax.numpy as jnp
from jax import lax
from jax.experimental import pallas as pl
from jax.experimental.pallas import tpu as pltpu
from jax.experimental.pallas import tpu_sc as plsc

D = 768
NW = 32


@functools.lru_cache(maxsize=None)
def _build(batch: int, seqlen: int):
    b_per_w = batch // NW
    assert batch % NW == 0 and b_per_w % 8 == 0

    mesh = plsc.VectorSubcoreMesh(core_axis_name="c", subcore_axis_name="s")

    nbuf = 4
    n_main = seqlen - (seqlen % nbuf)

    @functools.partial(
        pl.kernel,
        mesh=mesh,
        out_type=jax.ShapeDtypeStruct((seqlen, batch, D), jnp.float32),
        compiler_params=pltpu.CompilerParams(use_tc_tiling_on_sc=True),
        scratch_types=[
            pltpu.VMEM((seqlen * b_per_w,), jnp.int32),
            pltpu.VMEM((b_per_w, D), jnp.float32),
            pltpu.VMEM((b_per_w, D), jnp.float32),
            pltpu.VMEM((b_per_w, D), jnp.float32),
            pltpu.VMEM((b_per_w, D), jnp.float32),
            pltpu.SemaphoreType.DMA,
            pltpu.SemaphoreType.DMA,
            pltpu.SemaphoreType.DMA,
            pltpu.SemaphoreType.DMA,
        ],
    )
    def gather_kernel(ids_hbm, table_hbm, out_hbm, idx_v, buf0, buf1, buf2, buf3, sem0, sem1, sem2, sem3):
        wid = lax.axis_index("s") * 2 + lax.axis_index("c")
        pltpu.sync_copy(
            ids_hbm.at[pl.ds(wid * seqlen * b_per_w, seqlen * b_per_w)], idx_v
        )

        bufs = (buf0, buf1, buf2, buf3)
        sems = (sem0, sem1, sem2, sem3)

        def gather(s, b):
            pltpu.async_copy(
                table_hbm.at[idx_v.at[pl.ds(s * b_per_w, b_per_w)]],
                bufs[b],
                sems[b],
            )

        def wait(b):
            pltpu.make_async_copy(
                table_hbm.at[idx_v.at[pl.ds(0, b_per_w)]], bufs[b], sems[b]
            ).wait()

        def write(s, b):
            pltpu.sync_copy(bufs[b], out_hbm.at[s, pl.ds(wid * b_per_w, b_per_w)])

        for b in range(nbuf):
            gather(b, b)

        @pl.loop(0, n_main, step=nbuf)
        def _(s):
            for b in range(nbuf):
                sg = s + b
                wait(b)
                write(sg, b)

                @pl.when(sg + nbuf < seqlen)
                def _():
                    gather(sg + nbuf, b)

        for t in range(n_main, seqlen):
            wait(t % nbuf)
            write(t, t % nbuf)

    return gather_kernel


def kernel(input_ids, token_embedding):
    b, s = input_ids.shape
    ids = (
        input_ids.astype(jnp.int32)
        .reshape(NW, b // NW, s)
        .transpose(0, 2, 1)
        .reshape(-1)
    )
    out = _build(b, s)(ids, token_embedding)
    return out.transpose(1, 0, 2)


# speedup vs baseline: 1.0135x; 1.0135x over previous
import functools

import jax
import jax.numpy as jnp
from jax import lax
from jax.experimental import pallas as pl
from jax.experimental.pallas import tpu as pltpu
from jax.experimental.pallas import tpu_sc as plsc

D = 768
NW = 32


@functools.lru_cache(maxsize=None)
def _build(batch: int, seqlen: int):
    b_per_w = batch // NW
    assert batch % NW == 0 and b_per_w % 8 == 0

    mesh = plsc.VectorSubcoreMesh(core_axis_name="c", subcore_axis_name="s")

    n_main = seqlen - (seqlen % 2)

    @functools.partial(
        pl.kernel,
        mesh=mesh,
        out_type=jax.ShapeDtypeStruct((seqlen, batch, D), jnp.float32),
        compiler_params=pltpu.CompilerParams(use_tc_tiling_on_sc=True),
        scratch_types=[
            pltpu.VMEM((seqlen * b_per_w,), jnp.int32),
            pltpu.VMEM((b_per_w, D), jnp.float32),
            pltpu.VMEM((b_per_w, D), jnp.float32),
            pltpu.SemaphoreType.DMA,
            pltpu.SemaphoreType.DMA,
            pltpu.SemaphoreType.DMA,
            pltpu.SemaphoreType.DMA,
        ],
    )
    def gather_kernel(ids_hbm, table_hbm, out_hbm, idx_v, buf0, buf1, sem0, sem1, wsem0, wsem1):
        wid = lax.axis_index("s") * 2 + lax.axis_index("c")
        pltpu.sync_copy(
            ids_hbm.at[pl.ds(wid * seqlen * b_per_w, seqlen * b_per_w)], idx_v
        )

        bufs = (buf0, buf1)
        sems = (sem0, sem1)
        wsems = (wsem0, wsem1)

        def gather(s, b):
            pltpu.async_copy(
                table_hbm.at[idx_v.at[pl.ds(s * b_per_w, b_per_w)]],
                bufs[b],
                sems[b],
            )

        def wait(b):
            pltpu.make_async_copy(
                table_hbm.at[idx_v.at[pl.ds(0, b_per_w)]], bufs[b], sems[b]
            ).wait()

        def write_async(s, b):
            pltpu.async_copy(
                bufs[b], out_hbm.at[s, pl.ds(wid * b_per_w, b_per_w)], wsems[b]
            )

        def wait_write(b):
            pltpu.make_async_copy(
                bufs[b], out_hbm.at[0, pl.ds(wid * b_per_w, b_per_w)], wsems[b]
            ).wait()

        gather(0, 0)
        gather(1, 1)

        @pl.loop(0, n_main, step=2)
        def _(s):
            for b in range(2):
                sg = s + b
                wait(b)
                write_async(sg, b)

                @pl.when(sg + 2 < seqlen)
                def _():
                    wait_write(b)
                    gather(sg + 2, b)

        if n_main < seqlen:
            wait(0)
            write_async(seqlen - 1, 0)
        for b in range(2):
            wait_write(b)

    return gather_kernel


def kernel(input_ids, token_embedding):
    b, s = input_ids.shape
    ids = (
        input_ids.astype(jnp.int32)
        .reshape(NW, b // NW, s)
        .transpose(0, 2, 1)
        .reshape(-1)
    )
    out = _build(b, s)(ids, token_embedding)
    return out.transpose(1, 0, 2)


# speedup vs baseline: 1.0135x; 1.0000x over previous
import functools

import jax
import jax.numpy as jnp
from jax import lax
from jax.experimental import pallas as pl
from jax.experimental.pallas import tpu as pltpu
from jax.experimental.pallas import tpu_sc as plsc

D = 768
NW = 32


@functools.lru_cache(maxsize=None)
def _build(batch: int, seqlen: int):
    b_per_w = batch // NW
    assert batch % NW == 0 and b_per_w % 8 == 0

    mesh = plsc.VectorSubcoreMesh(core_axis_name="c", subcore_axis_name="s")

    n_main = seqlen - (seqlen % 2)

    @functools.partial(
        pl.kernel,
        mesh=mesh,
        out_type=jax.ShapeDtypeStruct((seqlen, batch, D), jnp.float32),
        compiler_params=pltpu.CompilerParams(use_tc_tiling_on_sc=True),
        scratch_types=[
            pltpu.VMEM((seqlen * b_per_w,), jnp.int32),
            pltpu.VMEM((b_per_w, D), jnp.float32),
            pltpu.VMEM((b_per_w, D), jnp.float32),
            pltpu.SemaphoreType.DMA,
            pltpu.SemaphoreType.DMA,
        ],
    )
    def gather_kernel(ids_hbm, table_hbm, out_hbm, idx_v, buf0, buf1, sem0, sem1):
        wid = lax.axis_index("s") * 2 + lax.axis_index("c")
        pltpu.sync_copy(
            ids_hbm.at[pl.ds(wid * seqlen * b_per_w, seqlen * b_per_w)], idx_v
        )

        bufs = (buf0, buf1)
        sems = (sem0, sem1)

        def gather(s, b):
            pltpu.async_copy(
                table_hbm.at[idx_v.at[pl.ds(s * b_per_w, b_per_w)]],
                bufs[b],
                sems[b],
            )

        def wait(b):
            pltpu.make_async_copy(
                table_hbm.at[idx_v.at[pl.ds(0, b_per_w)]], bufs[b], sems[b]
            ).wait()

        def write(s, b):
            pltpu.sync_copy(bufs[b], out_hbm.at[s, pl.ds(wid * b_per_w, b_per_w)])

        gather(0, 0)
        gather(1, 1)

        @pl.loop(0, n_main, step=2)
        def _(s):
            for b in range(2):
                sg = s + b
                wait(b)
                write(sg, b)

                @pl.when(sg + 2 < seqlen)
                def _():
                    gather(sg + 2, b)

        if n_main < seqlen:
            wait(0)
            write(seqlen - 1, 0)

    return gather_kernel


def kernel(input_ids, token_embedding):
    b, s = input_ids.shape
    ids = (
        input_ids.astype(jnp.int32)
        .reshape(NW, b // NW, s)
        .transpose(0, 2, 1)
        .reshape(-1)
    )
    out = _build(b, s)(ids, token_embedding)
    return out.transpose(1, 0, 2)
